# baseline (speedup 1.0000x reference)
import functools

import jax
import jax.numpy as jnp
from jax import lax
from jax.experimental import pallas as pl
from jax.experimental.pallas import tpu as pltpu
from jax.experimental.pallas import tpu_sc as plsc

NC = 2
NS = 16
NW = NC * NS

R = 4


def _sc_gather(x, table):
    B0, B1 = x.shape
    _, D = table.shape
    rows_w = B0 // NW
    n_chunks = rows_w // R
    mesh = plsc.VectorSubcoreMesh(core_axis_name="c", subcore_axis_name="s")

    @functools.partial(
        pl.kernel,
        mesh=mesh,
        compiler_params=pltpu.CompilerParams(use_tc_tiling_on_sc=False),
        out_type=jax.ShapeDtypeStruct((B0, B1, D), jnp.float32),
        scratch_types=[
            pltpu.VMEM((rows_w, B1), jnp.int32),
            pltpu.VMEM((2, R, B1, D), jnp.float32),
            pltpu.SemaphoreType.DMA,
            pltpu.SemaphoreType.DMA,
        ],
    )
    def k(x_hbm, table_hbm, out_hbm, idx_v, rows_v, gsem, ssem):
        wid = lax.axis_index("s") * NC + lax.axis_index("c")
        wrow = wid * rows_w

        pltpu.sync_copy(x_hbm.at[pl.ds(wrow, rows_w)], idx_v)

        def fire_gathers(c, b):
            return [
                pltpu.async_copy(
                    table_hbm.at[idx_v.at[c * R + r]],
                    rows_v.at[b].at[r],
                    gsem,
                )
                for r in range(R)
            ]

        def wait_gathers(c, b):
            for r in range(R):
                pltpu.make_async_copy(
                    table_hbm.at[idx_v.at[c * R + r]],
                    rows_v.at[b].at[r],
                    gsem,
                ).wait()

        def store(c, b):
            return pltpu.async_copy(
                rows_v.at[b], out_hbm.at[pl.ds(wrow + c * R, R)], ssem
            )

        def wait_store(c, b):
            pltpu.make_async_copy(
                rows_v.at[b], out_hbm.at[pl.ds(wrow + c * R, R)], ssem
            ).wait()

        for h in fire_gathers(0, 0):
            h.wait()
        store(0, 0)
        fire_gathers(1, 1)

        def pair_body(p, carry):
            for b, i in ((1, 2 * p + 1), (0, 2 * p + 2)):
                wait_gathers(i, b)
                wait_store(i - 1, 1 - b)
                fire_gathers(i + 1, 1 - b)
                store(i, b)
            return carry

        lax.fori_loop(0, (n_chunks - 2) // 2, pair_body, 0)

        last = n_chunks - 1
        lb = last % 2
        wait_gathers(last, lb)
        wait_store(last - 1, 1 - lb)
        store(last, lb).wait()

    return k(x, table)


def kernel(x, table):
    return _sc_gather(x, table)
